# baseline (device time: 71543 ns/iter reference)
import jax
import jax.numpy as jnp
from jax import lax
from jax.experimental import pallas as pl
from jax.experimental.pallas import tpu as pltpu

N_DEV = 16


def kernel(x, w_mat, scale_x, scale_w):
    m_per, k_dim = x.shape
    _, n_dim = w_mat.shape
    n_per = n_dim // N_DEV
    m_glob = N_DEV * m_per

    def body(x_ref, w_ref, sx_ref, sw_ref, out_ref,
             w_buf, xb, wb, sbuf, rbuf, dma_sems, send_sems, recv_sems):
        my = lax.axis_index("i")

        barrier = pltpu.get_barrier_semaphore()
        for k in range(1, N_DEV):
            pl.semaphore_signal(
                barrier, inc=1,
                device_id=((my + k) % N_DEV,),
                device_id_type=pl.DeviceIdType.MESH,
            )
        pl.semaphore_wait(barrier, N_DEV - 1)

        xb[...] = x_ref[...].astype(jnp.bfloat16)
        s_val = sx_ref[0] * sw_ref[0]

        def w_dma(j, slot):
            col = ((my + j) % N_DEV) * n_per
            return pltpu.make_async_copy(
                w_ref.at[:, pl.ds(col, n_per)],
                w_buf.at[slot],
                dma_sems.at[slot],
            )

        j_order = [8, 9, 10, 11, 4, 12, 5, 13, 6, 14, 7, 15, 1, 2, 3, 0]
        cur = w_dma(j_order[0], 0)
        cur.start()
        sends = []
        for t, j in enumerate(j_order):
            if t + 1 < N_DEV:
                nxt = w_dma(j_order[t + 1], (t + 1) % 2)
                nxt.start()
            cur.wait()
            wb[...] = w_buf[t % 2].astype(jnp.bfloat16)
            chunk = jnp.dot(
                xb[...], wb[...], preferred_element_type=jnp.float32
            ) * s_val
            if j == 0:
                out_ref[pl.ds(my * m_per, m_per), :] = chunk
            else:
                sbuf[j, :, :] = chunk.astype(jnp.bfloat16)
                rdma = pltpu.make_async_remote_copy(
                    src_ref=sbuf.at[j],
                    dst_ref=rbuf.at[j],
                    send_sem=send_sems.at[j],
                    recv_sem=recv_sems.at[j],
                    device_id=((my + j) % N_DEV,),
                    device_id_type=pl.DeviceIdType.MESH,
                )
                rdma.start()
                sends.append(rdma)
            if t + 1 < N_DEV:
                cur = nxt

        for k in range(1, N_DEV):
            src_dev = (my - k) % N_DEV
            recv = pltpu.make_async_remote_copy(
                src_ref=sbuf.at[k],
                dst_ref=rbuf.at[k],
                send_sem=send_sems.at[k],
                recv_sem=recv_sems.at[k],
                device_id=(src_dev,),
                device_id_type=pl.DeviceIdType.MESH,
            )
            recv.wait_recv()
            out_ref[pl.ds(src_dev * m_per, m_per), :] = (
                rbuf[k].astype(jnp.float32)
            )
        for rdma in sends:
            rdma.wait_send()

    return pl.pallas_call(
        body,
        out_shape=jax.ShapeDtypeStruct((m_glob, n_per), jnp.float32),
        in_specs=[
            pl.BlockSpec(memory_space=pltpu.VMEM),
            pl.BlockSpec(memory_space=pl.ANY),
            pl.BlockSpec(memory_space=pltpu.SMEM),
            pl.BlockSpec(memory_space=pltpu.SMEM),
        ],
        out_specs=pl.BlockSpec(memory_space=pltpu.VMEM),
        scratch_shapes=[
            pltpu.VMEM((2, k_dim, n_per), jnp.float32),
            pltpu.VMEM((m_per, k_dim), jnp.bfloat16),
            pltpu.VMEM((k_dim, n_per), jnp.bfloat16),
            pltpu.VMEM((N_DEV, m_per, n_per), jnp.bfloat16),
            pltpu.VMEM((N_DEV, m_per, n_per), jnp.bfloat16),
            pltpu.SemaphoreType.DMA((2,)),
            pltpu.SemaphoreType.DMA((N_DEV,)),
            pltpu.SemaphoreType.DMA((N_DEV,)),
        ],
        compiler_params=pltpu.CompilerParams(
            collective_id=0, vmem_limit_bytes=100 * 1024 * 1024,
        ),
    )(x, w_mat, scale_x, scale_w)


# device time: 70761 ns/iter; 1.0111x vs baseline; 1.0111x over previous
import jax
import jax.numpy as jnp
from jax import lax
from jax.experimental import pallas as pl
from jax.experimental.pallas import tpu as pltpu

N_DEV = 16


def kernel(x, w_mat, scale_x, scale_w):
    m_per, k_dim = x.shape
    _, n_dim = w_mat.shape
    n_per = n_dim // N_DEV
    m_glob = N_DEV * m_per

    def body(x_ref, w_ref, sx_ref, sw_ref, out_ref,
             w_buf, xb, wb, sbuf, rbuf, dma_sems, send_sems, recv_sems):
        my = lax.axis_index("i")

        barrier = pltpu.get_barrier_semaphore()
        for k in range(1, N_DEV):
            pl.semaphore_signal(
                barrier, inc=1,
                device_id=((my + k) % N_DEV,),
                device_id_type=pl.DeviceIdType.MESH,
            )
        pl.semaphore_wait(barrier, N_DEV - 1)

        xb[...] = x_ref[...].astype(jnp.bfloat16)
        s_val = sx_ref[0] * sw_ref[0]

        def w_dma(j, slot):
            col = ((my + j) % N_DEV) * n_per
            return pltpu.make_async_copy(
                w_ref.at[:, pl.ds(col, n_per)],
                w_buf.at[slot],
                dma_sems.at[slot],
            )

        j_order = [8, 9, 10, 11, 4, 12, 5, 13, 6, 14, 7, 15, 1, 2, 3, 0]
        n_slots = 3
        for t in range(2):
            w_dma(j_order[t], t % n_slots).start()
        sends = []
        for t, j in enumerate(j_order):
            if t + 2 < N_DEV:
                w_dma(j_order[t + 2], (t + 2) % n_slots).start()
            w_dma(j, t % n_slots).wait()
            wb[...] = w_buf[t % n_slots].astype(jnp.bfloat16)
            chunk = jnp.dot(
                xb[...], wb[...], preferred_element_type=jnp.float32
            ) * s_val
            if j == 0:
                out_ref[pl.ds(my * m_per, m_per), :] = chunk
            else:
                sbuf[j, :, :] = chunk.astype(jnp.bfloat16)
                rdma = pltpu.make_async_remote_copy(
                    src_ref=sbuf.at[j],
                    dst_ref=rbuf.at[j],
                    send_sem=send_sems.at[j],
                    recv_sem=recv_sems.at[j],
                    device_id=((my + j) % N_DEV,),
                    device_id_type=pl.DeviceIdType.MESH,
                )
                rdma.start()
                sends.append(rdma)

        for k in range(1, N_DEV):
            src_dev = (my - k) % N_DEV
            recv = pltpu.make_async_remote_copy(
                src_ref=sbuf.at[k],
                dst_ref=rbuf.at[k],
                send_sem=send_sems.at[k],
                recv_sem=recv_sems.at[k],
                device_id=(src_dev,),
                device_id_type=pl.DeviceIdType.MESH,
            )
            recv.wait_recv()
            out_ref[pl.ds(src_dev * m_per, m_per), :] = (
                rbuf[k].astype(jnp.float32)
            )
        for rdma in sends:
            rdma.wait_send()

    return pl.pallas_call(
        body,
        out_shape=jax.ShapeDtypeStruct((m_glob, n_per), jnp.float32),
        in_specs=[
            pl.BlockSpec(memory_space=pltpu.VMEM),
            pl.BlockSpec(memory_space=pl.ANY),
            pl.BlockSpec(memory_space=pltpu.SMEM),
            pl.BlockSpec(memory_space=pltpu.SMEM),
        ],
        out_specs=pl.BlockSpec(memory_space=pltpu.VMEM),
        scratch_shapes=[
            pltpu.VMEM((3, k_dim, n_per), jnp.float32),
            pltpu.VMEM((m_per, k_dim), jnp.bfloat16),
            pltpu.VMEM((k_dim, n_per), jnp.bfloat16),
            pltpu.VMEM((N_DEV, m_per, n_per), jnp.bfloat16),
            pltpu.VMEM((N_DEV, m_per, n_per), jnp.bfloat16),
            pltpu.SemaphoreType.DMA((3,)),
            pltpu.SemaphoreType.DMA((N_DEV,)),
            pltpu.SemaphoreType.DMA((N_DEV,)),
        ],
        compiler_params=pltpu.CompilerParams(
            collective_id=0, vmem_limit_bytes=100 * 1024 * 1024,
        ),
    )(x, w_mat, scale_x, scale_w)
